# baseline (speedup 1.0000x reference)
import functools

import jax
import jax.numpy as jnp
from jax import lax
from jax.experimental import pallas as pl
from jax.experimental.pallas import tpu as pltpu
from jax.experimental.pallas import tpu_sc as plsc

B = 2048
NUM_HIST = 20
NUM_OBS = 512
HIST_DIM = 10240
DIM = 128
N_HEAD = 4
CODE_DIM = 32
K = 8192

BB_A = 128
NB_A = B // BB_A
KC = 2048
BB_C = 512
NB_C = B // BB_C

SC_NC = 2
SC_NS = 16
SC_NW = SC_NC * SC_NS
ROWS_PER_W = (B * N_HEAD) // SC_NW
GATHER_CHUNK = 128
GATHER_W = 128
QROWS_PER_W = B // SC_NW

_NT = (((1,), (1,)), ((), ()))


def _elu(x):
    return jnp.where(x > 0, x, jnp.exp(x) - 1.0)


def _adapt_vq_body(hist_ref, w1_ref, b1_ref, w2_ref, b2_ref, w3_ref, b3_ref,
                   cb_ref, cbt_ref, latent_ref, idx_ref, cbpad_ref, z_scr):
    gi = pl.program_id(0)

    @pl.when(gi == 0)
    def _():
        cbpad_ref[:, 0:CODE_DIM] = cb_ref[...]

    rows = N_HEAD * BB_A

    h = jnp.zeros((BB_A, 2 * DIM), jnp.float32)
    for t in range(NUM_HIST):
        h = h + jnp.dot(hist_ref[t, :, :],
                        w1_ref[t * NUM_OBS:(t + 1) * NUM_OBS, :],
                        preferred_element_type=jnp.float32)
    h = _elu(h + b1_ref[...])
    h = _elu(jnp.dot(h, w2_ref[...], preferred_element_type=jnp.float32)
             + b2_ref[...])
    lat = (jnp.dot(h, w3_ref[...], preferred_element_type=jnp.float32)
           + b3_ref[...])
    latent_ref[...] = lat

    z = jnp.concatenate(
        [lat[:, i * CODE_DIM:(i + 1) * CODE_DIM] for i in range(N_HEAD)],
        axis=0)
    z_scr[gi % 2] = z

    zp = z_scr[(gi + 1) % 2]
    z2 = jnp.sum(zp * zp, axis=1, keepdims=True)
    zm2 = zp * -2.0
    cbt = cbt_ref[...]
    cb2 = jnp.sum(cbt * cbt, axis=0)

    iota = lax.broadcasted_iota(jnp.int32, (rows, KC), 1)
    best_val = jnp.full((rows,), jnp.inf, dtype=jnp.float32)
    best_idx = jnp.zeros((rows,), dtype=jnp.int32)
    for c in range(K // KC):
        s = jnp.dot(zm2, cbt[:, c * KC:(c + 1) * KC],
                    preferred_element_type=jnp.float32)
        d = (z2 + s) + cb2[None, c * KC:(c + 1) * KC]
        m = jnp.min(d, axis=1)
        cand = jnp.min(jnp.where(d == m[:, None], iota, K), axis=1)
        upd = m < best_val
        best_idx = jnp.where(upd, cand + c * KC, best_idx)
        best_val = jnp.minimum(best_val, m)
    idx_ref[jnp.maximum(gi - 1, 0), 0, :] = best_idx


def _actor_pre_body(obs_ref, wa1o_ref, ba1_ref, pre_ref):
    pre_ref[...] = (jnp.dot(obs_ref[...], wa1o_ref[...],
                            preferred_element_type=jnp.float32) + ba1_ref[...])


def _actor_body(pre_ref, lat_ref, q_ref, wa1l_ref, wa2_ref, ba2_ref,
                wa3_ref, ba3_ref, wa4_ref, ba4_ref, out_ref):
    lat = lat_ref[...]
    lq = lat + (q_ref[...] - lat)
    a = _elu(pre_ref[...]
             + jnp.dot(lq, wa1l_ref[...], preferred_element_type=jnp.float32))
    a = _elu(jnp.dot(a, wa2_ref[...], preferred_element_type=jnp.float32)
             + ba2_ref[...])
    a = _elu(jnp.dot(a, wa3_ref[...], preferred_element_type=jnp.float32)
             + ba3_ref[...])
    out_ref[...] = (jnp.dot(a, wa4_ref[...], preferred_element_type=jnp.float32)
                    + ba4_ref[...])


def _sc_gather_body(idx_hbm, table_hbm, out_hbm, idx_v, rows_v, out_v, sem):
    wid = lax.axis_index("s") * SC_NC + lax.axis_index("c")
    base = wid * ROWS_PER_W
    pltpu.sync_copy(idx_hbm.at[pl.ds(base, ROWS_PER_W)], idx_v)
    for j in range(ROWS_PER_W // GATHER_CHUNK):
        pltpu.async_copy(
            table_hbm.at[idx_v.at[pl.ds(j * GATHER_CHUNK, GATHER_CHUNK)]],
            rows_v.at[pl.ds(j * GATHER_CHUNK, GATHER_CHUNK)], sem).wait()

    def body(q, _):
        for h2 in range(N_HEAD * 2):
            h, t = h2 // 2, h2 % 2
            out_v[q, pl.ds(h * CODE_DIM + t * 16, 16)] = (
                rows_v[N_HEAD * q + h, pl.ds(t * 16, 16)])
        return _
    lax.fori_loop(0, QROWS_PER_W, body, None)
    pltpu.sync_copy(out_v, out_hbm.at[pl.ds(wid * QROWS_PER_W, QROWS_PER_W)])


def _adapt_vq(hist, W1, b1, W2, b2, W3, b3, cb, cbt):
    full = lambda shape: pl.BlockSpec(shape, lambda i: (0,) * len(shape))
    clamp = lambda i: jnp.minimum(i, NB_A - 1)
    return pl.pallas_call(
        _adapt_vq_body,
        grid=(NB_A + 1,),
        in_specs=[
            pl.BlockSpec((NUM_HIST, BB_A, NUM_OBS), lambda i: (0, clamp(i), 0)),
            full((HIST_DIM, 2 * DIM)),
            full((2 * DIM,)),
            full((2 * DIM, DIM)),
            full((DIM,)),
            full((DIM, DIM)),
            full((DIM,)),
            full((K, CODE_DIM)),
            full((CODE_DIM, K)),
        ],
        out_specs=[
            pl.BlockSpec((BB_A, DIM), lambda i: (clamp(i), 0)),
            pl.BlockSpec((NB_A, 1, N_HEAD * BB_A), lambda i: (0, 0, 0)),
            pl.BlockSpec((K, GATHER_W), lambda i: (0, 0)),
        ],
        out_shape=[
            jax.ShapeDtypeStruct((B, DIM), jnp.float32),
            jax.ShapeDtypeStruct((NB_A, 1, N_HEAD * BB_A), jnp.int32),
            jax.ShapeDtypeStruct((K, GATHER_W), jnp.float32),
        ],
        scratch_shapes=[pltpu.VMEM((2, N_HEAD * BB_A, CODE_DIM), jnp.float32)],
        compiler_params=pltpu.CompilerParams(
            dimension_semantics=("arbitrary",)),
    )(hist, W1, b1, W2, b2, W3, b3, cb, cbt)


def _sc_gather(idx_flat, table):
    mesh = plsc.VectorSubcoreMesh(core_axis_name="c", subcore_axis_name="s")
    kern = functools.partial(
        pl.kernel,
        mesh=mesh,
        out_type=jax.ShapeDtypeStruct((B, DIM), jnp.float32),
        scratch_types=[
            pltpu.VMEM((ROWS_PER_W,), jnp.int32),
            pltpu.VMEM((ROWS_PER_W, GATHER_W), jnp.float32),
            pltpu.VMEM((QROWS_PER_W, DIM), jnp.float32),
            pltpu.SemaphoreType.DMA,
        ],
    )(_sc_gather_body)
    return kern(idx_flat, table)


def _actor_pre(obs, Wa1o, ba1):
    full = lambda shape: pl.BlockSpec(shape, lambda i: (0,) * len(shape))
    return pl.pallas_call(
        _actor_pre_body,
        grid=(NB_C,),
        in_specs=[
            pl.BlockSpec((BB_C, 512), lambda i: (i, 0)),
            full((512, 512)),
            full((512,)),
        ],
        out_specs=pl.BlockSpec((BB_C, 512), lambda i: (i, 0)),
        out_shape=jax.ShapeDtypeStruct((B, 512), jnp.float32),
        compiler_params=pltpu.CompilerParams(
            dimension_semantics=("arbitrary",)),
    )(obs, Wa1o, ba1)


def _actor(pre, latent, quant, Wa1l, Wa2, ba2, Wa3, ba3, Wa4, ba4):
    full = lambda shape: pl.BlockSpec(shape, lambda i: (0,) * len(shape))
    return pl.pallas_call(
        _actor_body,
        grid=(NB_C,),
        in_specs=[
            pl.BlockSpec((BB_C, 512), lambda i: (i, 0)),
            pl.BlockSpec((BB_C, DIM), lambda i: (i, 0)),
            pl.BlockSpec((BB_C, DIM), lambda i: (i, 0)),
            full((DIM, 512)),
            full((512, 256)),
            full((256,)),
            full((256, 128)),
            full((128,)),
            full((128, 12)),
            full((12,)),
        ],
        out_specs=pl.BlockSpec((BB_C, 12), lambda i: (i, 0)),
        out_shape=jax.ShapeDtypeStruct((B, 12), jnp.float32),
        compiler_params=pltpu.CompilerParams(
            dimension_semantics=("arbitrary",)),
    )(pre, latent, quant, Wa1l, Wa2, ba2, Wa3, ba3, Wa4, ba4)


def kernel(obs, observation_history, W1, b1, W2, b2, W3, b3, codebook,
           Wa1, ba1, Wa2, ba2, Wa3, ba3, Wa4, ba4):
    hist_t = jnp.transpose(observation_history, (1, 0, 2))
    latent, idx_blocks, table = _adapt_vq(hist_t, W1, b1, W2, b2,
                                          W3, b3, codebook, codebook.T)
    idx_flat = (idx_blocks.reshape(NB_A, N_HEAD, BB_A)
                .transpose(0, 2, 1).reshape(B * N_HEAD))
    quant = _sc_gather(idx_flat, table)
    pre = _actor_pre(obs, Wa1[:512], ba1)
    return _actor(pre, latent, quant, Wa1[512:], Wa2, ba2, Wa3, ba3, Wa4, ba4)


# speedup vs baseline: 1.0605x; 1.0605x over previous
import functools

import jax
import jax.numpy as jnp
from jax import lax
from jax.experimental import pallas as pl
from jax.experimental.pallas import tpu as pltpu
from jax.experimental.pallas import tpu_sc as plsc

B = 2048
NUM_HIST = 20
NUM_OBS = 512
HIST_DIM = 10240
DIM = 128
N_HEAD = 4
CODE_DIM = 32
K = 8192

BB_A = 128
NB_A = B // BB_A
KC = 2048
BB_C = 512
NB_C = B // BB_C

SC_NC = 2
SC_NS = 16
SC_NW = SC_NC * SC_NS
ROWS_PER_W = (B * N_HEAD) // SC_NW
GATHER_CHUNK = 128
GATHER_W = 128
QROWS_PER_W = B // SC_NW

_NT = (((1,), (1,)), ((), ()))


def _elu(x):
    return jnp.where(x > 0, x, jnp.exp(x) - 1.0)


def _adapt_vq_body(hist_ref, w1_ref, b1_ref, w2_ref, b2_ref, w3_ref, b3_ref,
                   cb_ref, cbt_ref, latent_ref, idx_ref, cbpad_ref):
    @pl.when(pl.program_id(0) == 0)
    def _():
        cbpad_ref[:, 0:CODE_DIM] = cb_ref[...]

    h = jnp.zeros((BB_A, 2 * DIM), jnp.float32)
    for t in range(NUM_HIST):
        h = h + jnp.dot(hist_ref[t, :, :],
                        w1_ref[t * NUM_OBS:(t + 1) * NUM_OBS, :],
                        preferred_element_type=jnp.float32)
    h = _elu(h + b1_ref[...])
    h = _elu(jnp.dot(h, w2_ref[...], preferred_element_type=jnp.float32)
             + b2_ref[...])
    lat = (jnp.dot(h, w3_ref[...], preferred_element_type=jnp.float32)
           + b3_ref[...])
    latent_ref[...] = lat

    z = jnp.concatenate(
        [lat[:, i * CODE_DIM:(i + 1) * CODE_DIM] for i in range(N_HEAD)],
        axis=0)
    z2 = jnp.sum(z * z, axis=1, keepdims=True)
    zm2 = z * -2.0
    cbt = cbt_ref[...]
    cb2 = jnp.sum(cbt * cbt, axis=0)

    rows = N_HEAD * BB_A
    iota = lax.broadcasted_iota(jnp.int32, (rows, KC), 1)
    best_val = jnp.full((rows,), jnp.inf, dtype=jnp.float32)
    best_idx = jnp.zeros((rows,), dtype=jnp.int32)
    for c in range(K // KC):
        s = jnp.dot(zm2, cbt[:, c * KC:(c + 1) * KC],
                    preferred_element_type=jnp.float32)
        d = (z2 + s) + cb2[None, c * KC:(c + 1) * KC]
        m = jnp.min(d, axis=1)
        cand = jnp.min(jnp.where(d == m[:, None], iota, K), axis=1)
        upd = m < best_val
        best_idx = jnp.where(upd, cand + c * KC, best_idx)
        best_val = jnp.minimum(best_val, m)
    idx_ref[...] = best_idx.reshape(1, 1, rows)


def _actor_body(obs_ref, lat_ref, q_ref, wa1_ref, ba1_ref, wa2_ref, ba2_ref,
                wa3_ref, ba3_ref, wa4_ref, ba4_ref, out_ref):
    lat = lat_ref[...]
    lq = lat + (q_ref[...] - lat)
    a = jnp.concatenate([obs_ref[...], lq], axis=1)
    a = _elu(jnp.dot(a, wa1_ref[...], preferred_element_type=jnp.float32)
             + ba1_ref[...])
    a = _elu(jnp.dot(a, wa2_ref[...], preferred_element_type=jnp.float32)
             + ba2_ref[...])
    a = _elu(jnp.dot(a, wa3_ref[...], preferred_element_type=jnp.float32)
             + ba3_ref[...])
    out_ref[...] = (jnp.dot(a, wa4_ref[...], preferred_element_type=jnp.float32)
                    + ba4_ref[...])


def _sc_gather_body(idx_hbm, table_hbm, out_hbm, idx_v, rows_v, out_v, sem):
    wid = lax.axis_index("s") * SC_NC + lax.axis_index("c")
    base = wid * ROWS_PER_W
    pltpu.sync_copy(idx_hbm.at[pl.ds(base, ROWS_PER_W)], idx_v)
    for j in range(ROWS_PER_W // GATHER_CHUNK):
        pltpu.async_copy(
            table_hbm.at[idx_v.at[pl.ds(j * GATHER_CHUNK, GATHER_CHUNK)]],
            rows_v.at[pl.ds(j * GATHER_CHUNK, GATHER_CHUNK)], sem).wait()

    def body(q, _):
        for h2 in range(N_HEAD * 2):
            h, t = h2 // 2, h2 % 2
            out_v[q, pl.ds(h * CODE_DIM + t * 16, 16)] = (
                rows_v[N_HEAD * q + h, pl.ds(t * 16, 16)])
        return _
    lax.fori_loop(0, QROWS_PER_W, body, None)
    pltpu.sync_copy(out_v, out_hbm.at[pl.ds(wid * QROWS_PER_W, QROWS_PER_W)])


def _adapt_vq(hist, W1, b1, W2, b2, W3, b3, cb, cbt):
    full = lambda shape: pl.BlockSpec(shape, lambda i: (0,) * len(shape))
    return pl.pallas_call(
        _adapt_vq_body,
        grid=(NB_A,),
        in_specs=[
            pl.BlockSpec((NUM_HIST, BB_A, NUM_OBS), lambda i: (0, i, 0)),
            full((HIST_DIM, 2 * DIM)),
            full((2 * DIM,)),
            full((2 * DIM, DIM)),
            full((DIM,)),
            full((DIM, DIM)),
            full((DIM,)),
            full((K, CODE_DIM)),
            full((CODE_DIM, K)),
        ],
        out_specs=[
            pl.BlockSpec((BB_A, DIM), lambda i: (i, 0)),
            pl.BlockSpec((1, 1, N_HEAD * BB_A), lambda i: (i, 0, 0)),
            pl.BlockSpec((K, GATHER_W), lambda i: (0, 0)),
        ],
        out_shape=[
            jax.ShapeDtypeStruct((B, DIM), jnp.float32),
            jax.ShapeDtypeStruct((NB_A, 1, N_HEAD * BB_A), jnp.int32),
            jax.ShapeDtypeStruct((K, GATHER_W), jnp.float32),
        ],
        compiler_params=pltpu.CompilerParams(
            dimension_semantics=("arbitrary",)),
    )(hist, W1, b1, W2, b2, W3, b3, cb, cbt)


def _sc_gather(idx_flat, table):
    mesh = plsc.VectorSubcoreMesh(core_axis_name="c", subcore_axis_name="s")
    kern = functools.partial(
        pl.kernel,
        mesh=mesh,
        out_type=jax.ShapeDtypeStruct((B, DIM), jnp.float32),
        scratch_types=[
            pltpu.VMEM((ROWS_PER_W,), jnp.int32),
            pltpu.VMEM((ROWS_PER_W, GATHER_W), jnp.float32),
            pltpu.VMEM((QROWS_PER_W, DIM), jnp.float32),
            pltpu.SemaphoreType.DMA,
        ],
    )(_sc_gather_body)
    return kern(idx_flat, table)


def _actor(obs, latent, quant, Wa1, ba1, Wa2, ba2, Wa3, ba3, Wa4, ba4):
    full = lambda shape: pl.BlockSpec(shape, lambda i: (0,) * len(shape))
    return pl.pallas_call(
        _actor_body,
        grid=(NB_C,),
        in_specs=[
            pl.BlockSpec((BB_C, 512), lambda i: (i, 0)),
            pl.BlockSpec((BB_C, DIM), lambda i: (i, 0)),
            pl.BlockSpec((BB_C, DIM), lambda i: (i, 0)),
            full((512 + DIM, 512)),
            full((512,)),
            full((512, 256)),
            full((256,)),
            full((256, 128)),
            full((128,)),
            full((128, 12)),
            full((12,)),
        ],
        out_specs=pl.BlockSpec((BB_C, 12), lambda i: (i, 0)),
        out_shape=jax.ShapeDtypeStruct((B, 12), jnp.float32),
        compiler_params=pltpu.CompilerParams(
            dimension_semantics=("arbitrary",)),
    )(obs, latent, quant, Wa1, ba1, Wa2, ba2, Wa3, ba3, Wa4, ba4)


def kernel(obs, observation_history, W1, b1, W2, b2, W3, b3, codebook,
           Wa1, ba1, Wa2, ba2, Wa3, ba3, Wa4, ba4):
    hist_t = jnp.transpose(observation_history, (1, 0, 2))
    latent, idx_blocks, table = _adapt_vq(hist_t, W1, b1, W2, b2,
                                          W3, b3, codebook, codebook.T)
    idx_flat = (idx_blocks.reshape(NB_A, N_HEAD, BB_A)
                .transpose(0, 2, 1).reshape(B * N_HEAD))
    quant = _sc_gather(idx_flat, table)
    return _actor(obs, latent, quant, Wa1, ba1, Wa2, ba2, Wa3, ba3, Wa4, ba4)


# speedup vs baseline: 1.0786x; 1.0170x over previous
import functools

import jax
import jax.numpy as jnp
from jax import lax
from jax.experimental import pallas as pl
from jax.experimental.pallas import tpu as pltpu
from jax.experimental.pallas import tpu_sc as plsc

B = 2048
NUM_HIST = 20
NUM_OBS = 512
HIST_DIM = 10240
DIM = 128
N_HEAD = 4
CODE_DIM = 32
K = 8192

BB_A = 256
NB_A = B // BB_A
KC = 1024
BB_C = 512
NB_C = B // BB_C

SC_NC = 2
SC_NS = 16
SC_NW = SC_NC * SC_NS
ROWS_PER_W = (B * N_HEAD) // SC_NW
GATHER_CHUNK = 128
GATHER_W = 128
QROWS_PER_W = B // SC_NW

_NT = (((1,), (1,)), ((), ()))


def _elu(x):
    return jnp.where(x > 0, x, jnp.exp(x) - 1.0)


def _adapt_vq_body(hist_ref, w1_ref, b1_ref, w2_ref, b2_ref, w3_ref, b3_ref,
                   cb_ref, cbt_ref, latent_ref, idx_ref, cbpad_ref):
    @pl.when(pl.program_id(0) == 0)
    def _():
        cbpad_ref[:, 0:CODE_DIM] = cb_ref[...]

    h = jnp.zeros((BB_A, 2 * DIM), jnp.float32)
    for t in range(NUM_HIST):
        h = h + jnp.dot(hist_ref[t, :, :],
                        w1_ref[t * NUM_OBS:(t + 1) * NUM_OBS, :],
                        preferred_element_type=jnp.float32)
    h = _elu(h + b1_ref[...])
    h = _elu(jnp.dot(h, w2_ref[...], preferred_element_type=jnp.float32)
             + b2_ref[...])
    lat = (jnp.dot(h, w3_ref[...], preferred_element_type=jnp.float32)
           + b3_ref[...])
    latent_ref[...] = lat

    z = jnp.concatenate(
        [lat[:, i * CODE_DIM:(i + 1) * CODE_DIM] for i in range(N_HEAD)],
        axis=0)
    z2 = jnp.sum(z * z, axis=1, keepdims=True)
    zm2 = z * -2.0
    cbt = cbt_ref[...]
    cb2 = jnp.sum(cbt * cbt, axis=0)

    rows = N_HEAD * BB_A
    iota = lax.broadcasted_iota(jnp.int32, (rows, KC), 1)
    best_val = jnp.full((rows,), jnp.inf, dtype=jnp.float32)
    best_idx = jnp.zeros((rows,), dtype=jnp.int32)
    for c in range(K // KC):
        s = jnp.dot(zm2, cbt[:, c * KC:(c + 1) * KC],
                    preferred_element_type=jnp.float32)
        d = (z2 + s) + cb2[None, c * KC:(c + 1) * KC]
        m = jnp.min(d, axis=1)
        cand = jnp.min(jnp.where(d == m[:, None], iota, K), axis=1)
        upd = m < best_val
        best_idx = jnp.where(upd, cand + c * KC, best_idx)
        best_val = jnp.minimum(best_val, m)
    idx_ref[...] = best_idx.reshape(1, 1, rows)


def _actor_body(obs_ref, lat_ref, q_ref, wa1_ref, ba1_ref, wa2_ref, ba2_ref,
                wa3_ref, ba3_ref, wa4_ref, ba4_ref, out_ref):
    lat = lat_ref[...]
    lq = lat + (q_ref[...] - lat)
    a = jnp.concatenate([obs_ref[...], lq], axis=1)
    a = _elu(jnp.dot(a, wa1_ref[...], preferred_element_type=jnp.float32)
             + ba1_ref[...])
    a = _elu(jnp.dot(a, wa2_ref[...], preferred_element_type=jnp.float32)
             + ba2_ref[...])
    a = _elu(jnp.dot(a, wa3_ref[...], preferred_element_type=jnp.float32)
             + ba3_ref[...])
    out_ref[...] = (jnp.dot(a, wa4_ref[...], preferred_element_type=jnp.float32)
                    + ba4_ref[...])


def _sc_gather_body(idx_hbm, table_hbm, out_hbm, idx_v, rows_v, out_v, sem):
    wid = lax.axis_index("s") * SC_NC + lax.axis_index("c")
    base = wid * ROWS_PER_W
    pltpu.sync_copy(idx_hbm.at[pl.ds(base, ROWS_PER_W)], idx_v)
    for j in range(ROWS_PER_W // GATHER_CHUNK):
        pltpu.async_copy(
            table_hbm.at[idx_v.at[pl.ds(j * GATHER_CHUNK, GATHER_CHUNK)]],
            rows_v.at[pl.ds(j * GATHER_CHUNK, GATHER_CHUNK)], sem).wait()

    def body(q, _):
        for h2 in range(N_HEAD * 2):
            h, t = h2 // 2, h2 % 2
            out_v[q, pl.ds(h * CODE_DIM + t * 16, 16)] = (
                rows_v[N_HEAD * q + h, pl.ds(t * 16, 16)])
        return _
    lax.fori_loop(0, QROWS_PER_W, body, None)
    pltpu.sync_copy(out_v, out_hbm.at[pl.ds(wid * QROWS_PER_W, QROWS_PER_W)])


def _adapt_vq(hist, W1, b1, W2, b2, W3, b3, cb, cbt):
    full = lambda shape: pl.BlockSpec(shape, lambda i: (0,) * len(shape))
    return pl.pallas_call(
        _adapt_vq_body,
        grid=(NB_A,),
        in_specs=[
            pl.BlockSpec((NUM_HIST, BB_A, NUM_OBS), lambda i: (0, i, 0)),
            full((HIST_DIM, 2 * DIM)),
            full((2 * DIM,)),
            full((2 * DIM, DIM)),
            full((DIM,)),
            full((DIM, DIM)),
            full((DIM,)),
            full((K, CODE_DIM)),
            full((CODE_DIM, K)),
        ],
        out_specs=[
            pl.BlockSpec((BB_A, DIM), lambda i: (i, 0)),
            pl.BlockSpec((1, 1, N_HEAD * BB_A), lambda i: (i, 0, 0)),
            pl.BlockSpec((K, GATHER_W), lambda i: (0, 0)),
        ],
        out_shape=[
            jax.ShapeDtypeStruct((B, DIM), jnp.float32),
            jax.ShapeDtypeStruct((NB_A, 1, N_HEAD * BB_A), jnp.int32),
            jax.ShapeDtypeStruct((K, GATHER_W), jnp.float32),
        ],
        compiler_params=pltpu.CompilerParams(
            dimension_semantics=("arbitrary",)),
    )(hist, W1, b1, W2, b2, W3, b3, cb, cbt)


def _sc_gather(idx_flat, table):
    mesh = plsc.VectorSubcoreMesh(core_axis_name="c", subcore_axis_name="s")
    kern = functools.partial(
        pl.kernel,
        mesh=mesh,
        out_type=jax.ShapeDtypeStruct((B, DIM), jnp.float32),
        scratch_types=[
            pltpu.VMEM((ROWS_PER_W,), jnp.int32),
            pltpu.VMEM((ROWS_PER_W, GATHER_W), jnp.float32),
            pltpu.VMEM((QROWS_PER_W, DIM), jnp.float32),
            pltpu.SemaphoreType.DMA,
        ],
    )(_sc_gather_body)
    return kern(idx_flat, table)


def _actor(obs, latent, quant, Wa1, ba1, Wa2, ba2, Wa3, ba3, Wa4, ba4):
    full = lambda shape: pl.BlockSpec(shape, lambda i: (0,) * len(shape))
    return pl.pallas_call(
        _actor_body,
        grid=(NB_C,),
        in_specs=[
            pl.BlockSpec((BB_C, 512), lambda i: (i, 0)),
            pl.BlockSpec((BB_C, DIM), lambda i: (i, 0)),
            pl.BlockSpec((BB_C, DIM), lambda i: (i, 0)),
            full((512 + DIM, 512)),
            full((512,)),
            full((512, 256)),
            full((256,)),
            full((256, 128)),
            full((128,)),
            full((128, 12)),
            full((12,)),
        ],
        out_specs=pl.BlockSpec((BB_C, 12), lambda i: (i, 0)),
        out_shape=jax.ShapeDtypeStruct((B, 12), jnp.float32),
        compiler_params=pltpu.CompilerParams(
            dimension_semantics=("arbitrary",)),
    )(obs, latent, quant, Wa1, ba1, Wa2, ba2, Wa3, ba3, Wa4, ba4)


def kernel(obs, observation_history, W1, b1, W2, b2, W3, b3, codebook,
           Wa1, ba1, Wa2, ba2, Wa3, ba3, Wa4, ba4):
    hist_t = jnp.transpose(observation_history, (1, 0, 2))
    latent, idx_blocks, table = _adapt_vq(hist_t, W1, b1, W2, b2,
                                          W3, b3, codebook, codebook.T)
    idx_flat = (idx_blocks.reshape(NB_A, N_HEAD, BB_A)
                .transpose(0, 2, 1).reshape(B * N_HEAD))
    quant = _sc_gather(idx_flat, table)
    return _actor(obs, latent, quant, Wa1, ba1, Wa2, ba2, Wa3, ba3, Wa4, ba4)


# speedup vs baseline: 1.0787x; 1.0002x over previous
import functools

import jax
import jax.numpy as jnp
from jax import lax
from jax.experimental import pallas as pl
from jax.experimental.pallas import tpu as pltpu
from jax.experimental.pallas import tpu_sc as plsc

B = 2048
NUM_HIST = 20
NUM_OBS = 512
HIST_DIM = 10240
DIM = 128
N_HEAD = 4
CODE_DIM = 32
K = 8192

BB_A = 256
NB_A = B // BB_A
KC = 1024
BB_C = 512
NB_C = B // BB_C

SC_NC = 2
SC_NS = 16
SC_NW = SC_NC * SC_NS
ROWS_PER_W = (B * N_HEAD) // SC_NW
GATHER_CHUNK = 128
GATHER_W = 128
QROWS_PER_W = B // SC_NW


def _elu(x):
    return jnp.where(x > 0, x, jnp.exp(x) - 1.0)


def _adapt_vq_body(hist_ref, w1_ref, b1_ref, w2_ref, b2_ref, w3_ref, b3_ref,
                   cb_ref, cbt_ref, latent_ref, idx_ref, cbpad_ref):
    @pl.when(pl.program_id(0) == 0)
    def _():
        cbpad_ref[:, 0:CODE_DIM] = cb_ref[...]

    h = jnp.zeros((BB_A, 2 * DIM), jnp.float32)
    for t in range(NUM_HIST):
        h = h + jnp.dot(hist_ref[t, :, :],
                        w1_ref[t * NUM_OBS:(t + 1) * NUM_OBS, :],
                        preferred_element_type=jnp.float32)
    h = _elu(h + b1_ref[...])
    h = _elu(jnp.dot(h, w2_ref[...], preferred_element_type=jnp.float32)
             + b2_ref[...])
    lat = (jnp.dot(h, w3_ref[...], preferred_element_type=jnp.float32)
           + b3_ref[...])
    latent_ref[...] = lat

    z = jnp.concatenate(
        [lat[:, i * CODE_DIM:(i + 1) * CODE_DIM] for i in range(N_HEAD)],
        axis=0)
    z2 = jnp.sum(z * z, axis=1, keepdims=True)
    zm2 = z * -2.0
    cbt = cbt_ref[...]
    cb2 = jnp.sum(cbt * cbt, axis=0)

    rows = N_HEAD * BB_A
    iota = lax.broadcasted_iota(jnp.int32, (rows, KC), 1)
    best_val = jnp.full((rows,), jnp.inf, dtype=jnp.float32)
    best_idx = jnp.zeros((rows,), dtype=jnp.int32)
    for c in range(K // KC):
        s = jnp.dot(zm2, cbt[:, c * KC:(c + 1) * KC],
                    preferred_element_type=jnp.float32)
        d = (z2 + s) + cb2[None, c * KC:(c + 1) * KC]
        m = jnp.min(d, axis=1)
        cand = jnp.min(jnp.where(d == m[:, None], iota, K), axis=1)
        upd = m < best_val
        best_idx = jnp.where(upd, cand + c * KC, best_idx)
        best_val = jnp.minimum(best_val, m)
    idx_ref[...] = best_idx.reshape(1, 1, rows)


def _actor_body(obs_ref, lat_ref, q_ref, wa1_ref, ba1_ref, wa2_ref, ba2_ref,
                wa3_ref, ba3_ref, wa4_ref, ba4_ref, out_ref):
    lat = lat_ref[...]
    lq = lat + (q_ref[...] - lat)
    a = jnp.concatenate([obs_ref[...], lq], axis=1)
    a = _elu(jnp.dot(a, wa1_ref[...], preferred_element_type=jnp.float32)
             + ba1_ref[...])
    a = _elu(jnp.dot(a, wa2_ref[...], preferred_element_type=jnp.float32)
             + ba2_ref[...])
    a = _elu(jnp.dot(a, wa3_ref[...], preferred_element_type=jnp.float32)
             + ba3_ref[...])
    out_ref[...] = (jnp.dot(a, wa4_ref[...], preferred_element_type=jnp.float32)
                    + ba4_ref[...])


def _sc_gather_body(idx_hbm, table_hbm, out_hbm, idx_v, rows_v, out_v, sem):
    wid = lax.axis_index("s") * SC_NC + lax.axis_index("c")
    base = wid * ROWS_PER_W
    pltpu.sync_copy(idx_hbm.at[pl.ds(base, ROWS_PER_W)], idx_v)
    for j in range(ROWS_PER_W // GATHER_CHUNK):
        pltpu.async_copy(
            table_hbm.at[idx_v.at[pl.ds(j * GATHER_CHUNK, GATHER_CHUNK)]],
            rows_v.at[pl.ds(j * GATHER_CHUNK, GATHER_CHUNK)], sem).wait()

    def body(q, _):
        for h2 in range(N_HEAD * 2):
            h, t = h2 // 2, h2 % 2
            out_v[q, pl.ds(h * CODE_DIM + t * 16, 16)] = (
                rows_v[N_HEAD * q + h, pl.ds(t * 16, 16)])
        return _
    lax.fori_loop(0, QROWS_PER_W, body, None)
    pltpu.sync_copy(out_v, out_hbm.at[pl.ds(wid * QROWS_PER_W, QROWS_PER_W)])


def _adapt_vq(hist, W1, b1, W2, b2, W3, b3, cb, cbt):
    full = lambda shape: pl.BlockSpec(shape, lambda i: (0,) * len(shape))
    return pl.pallas_call(
        _adapt_vq_body,
        grid=(NB_A,),
        in_specs=[
            pl.BlockSpec((NUM_HIST, BB_A, NUM_OBS), lambda i: (0, i, 0)),
            full((HIST_DIM, 2 * DIM)),
            full((2 * DIM,)),
            full((2 * DIM, DIM)),
            full((DIM,)),
            full((DIM, DIM)),
            full((DIM,)),
            full((K, CODE_DIM)),
            full((CODE_DIM, K)),
        ],
        out_specs=[
            pl.BlockSpec((BB_A, DIM), lambda i: (i, 0)),
            pl.BlockSpec((1, 1, N_HEAD * BB_A), lambda i: (i, 0, 0)),
            pl.BlockSpec((K, GATHER_W), lambda i: (0, 0)),
        ],
        out_shape=[
            jax.ShapeDtypeStruct((B, DIM), jnp.float32),
            jax.ShapeDtypeStruct((NB_A, 1, N_HEAD * BB_A), jnp.int32),
            jax.ShapeDtypeStruct((K, GATHER_W), jnp.float32),
        ],
        compiler_params=pltpu.CompilerParams(
            dimension_semantics=("arbitrary",)),
    )(hist, W1, b1, W2, b2, W3, b3, cb, cbt)


def _sc_gather(idx_flat, table):
    mesh = plsc.VectorSubcoreMesh(core_axis_name="c", subcore_axis_name="s")
    kern = functools.partial(
        pl.kernel,
        mesh=mesh,
        out_type=jax.ShapeDtypeStruct((B, DIM), jnp.float32),
        scratch_types=[
            pltpu.VMEM((ROWS_PER_W,), jnp.int32),
            pltpu.VMEM((ROWS_PER_W, GATHER_W), jnp.float32),
            pltpu.VMEM((QROWS_PER_W, DIM), jnp.float32),
            pltpu.SemaphoreType.DMA,
        ],
    )(_sc_gather_body)
    return kern(idx_flat, table)


def _actor(obs, latent, quant, Wa1, ba1, Wa2, ba2, Wa3, ba3, Wa4, ba4):
    full = lambda shape: pl.BlockSpec(shape, lambda i: (0,) * len(shape))
    return pl.pallas_call(
        _actor_body,
        grid=(NB_C,),
        in_specs=[
            pl.BlockSpec((BB_C, 512), lambda i: (i, 0)),
            pl.BlockSpec((BB_C, DIM), lambda i: (i, 0)),
            pl.BlockSpec((BB_C, DIM), lambda i: (i, 0)),
            full((512 + DIM, 512)),
            full((512,)),
            full((512, 256)),
            full((256,)),
            full((256, 128)),
            full((128,)),
            full((128, 12)),
            full((12,)),
        ],
        out_specs=pl.BlockSpec((BB_C, 12), lambda i: (i, 0)),
        out_shape=jax.ShapeDtypeStruct((B, 12), jnp.float32),
        compiler_params=pltpu.CompilerParams(
            dimension_semantics=("arbitrary",)),
    )(obs, latent, quant, Wa1, ba1, Wa2, ba2, Wa3, ba3, Wa4, ba4)


def kernel(obs, observation_history, W1, b1, W2, b2, W3, b3, codebook,
           Wa1, ba1, Wa2, ba2, Wa3, ba3, Wa4, ba4):
    hist_t = jnp.transpose(observation_history, (1, 0, 2))
    latent, idx_blocks, table = _adapt_vq(hist_t, W1, b1, W2, b2,
                                          W3, b3, codebook, codebook.T)
    idx_flat = (idx_blocks.reshape(NB_A, N_HEAD, BB_A)
                .transpose(0, 2, 1).reshape(B * N_HEAD))
    quant = _sc_gather(idx_flat, table)
    return _actor(obs, latent, quant, Wa1, ba1, Wa2, ba2, Wa3, ba3, Wa4, ba4)
